# Initial kernel scaffold; baseline (speedup 1.0000x reference)
import jax
import jax.numpy as jnp
from jax import lax
from jax.experimental import pallas as pl
from jax.experimental.pallas import tpu as pltpu
from jax.experimental.pallas import tpu_sc as plsc

N = 10000
E = 320000
D = 128
NC = 2
NS = 16
NW = NC * NS
CHUNK = 128
CPW = 80
EPW = CHUNK * CPW
E_PAD = NW * EPW
N_ACC = N + 16
RPT = N_ACC // NS
BLK = 1000
NB = N // BLK


def _deg_body(col_hbm, hist_hbm, cidx_v, ones_v, zb_v, deg_sh):
    c = lax.axis_index("c")
    s = lax.axis_index("s")
    wid = s * NC + c
    pltpu.sync_copy(col_hbm.at[wid], cidx_v)

    def fill_ones(i, carry):
        ones_v[i, :] = jnp.ones((16,), jnp.float32)
        return carry

    lax.fori_loop(0, CHUNK, fill_ones, 0)

    def fill_zero(i, carry):
        zb_v[i, :] = jnp.zeros((16,), jnp.float32)
        return carry

    lax.fori_loop(0, RPT, fill_zero, 0)
    pltpu.sync_copy(zb_v, deg_sh.at[pl.ds(s * RPT, RPT)])
    plsc.subcore_barrier()

    def acc_step(j, carry):
        pltpu.sync_copy(ones_v, deg_sh.at[cidx_v.at[j]], add=True)
        return carry

    lax.fori_loop(0, CPW, acc_step, 0)
    plsc.subcore_barrier()
    pltpu.sync_copy(deg_sh.at[pl.ds(s * RPT, RPT)],
                    hist_hbm.at[c, pl.ds(s * RPT, RPT)])


_deg_kernel = pl.kernel(
    _deg_body,
    out_type=jax.ShapeDtypeStruct((NC, N_ACC, 16), jnp.float32),
    mesh=plsc.VectorSubcoreMesh(core_axis_name="c", subcore_axis_name="s"),
    scratch_types=[
        pltpu.VMEM((CPW, CHUNK), jnp.int32),
        pltpu.VMEM((CHUNK, 16), jnp.float32),
        pltpu.VMEM((RPT, 16), jnp.float32),
        pltpu.VMEM_SHARED((N_ACC, 16), jnp.float32),
    ],
)


def _prop_body(row_hbm, col_hbm, y_hbm, accs_hbm,
               ridx_v, cidx_v, rows_v, acc_sh, sem0, sem1):
    c = lax.axis_index("c")
    s = lax.axis_index("s")
    wid = s * NC + c
    pltpu.sync_copy(row_hbm.at[wid], ridx_v)
    pltpu.sync_copy(col_hbm.at[wid], cidx_v)

    def fill_zero(i, carry):
        for l in range(D // 16):
            rows_v[0, i, pl.ds(l * 16, 16)] = jnp.zeros((16,), jnp.float32)
        return carry

    lax.fori_loop(0, CHUNK, fill_zero, 0)
    base = s * RPT
    for k in range(RPT // CHUNK):
        pltpu.sync_copy(rows_v.at[0],
                        acc_sh.at[pl.ds(base + k * CHUNK, CHUNK)])
    rem = RPT % CHUNK
    pltpu.sync_copy(rows_v.at[0, pl.ds(0, rem), :],
                    acc_sh.at[pl.ds(base + RPT - rem, rem)])
    plsc.subcore_barrier()

    pltpu.async_copy(y_hbm.at[ridx_v.at[0]], rows_v.at[0], sem0)
    pltpu.async_copy(y_hbm.at[ridx_v.at[1]], rows_v.at[1], sem1)

    def step(i, carry):
        j0 = 2 * i
        pltpu.make_async_copy(y_hbm.at[ridx_v.at[j0]], rows_v.at[0],
                              sem0).wait()
        pltpu.sync_copy(rows_v.at[0], acc_sh.at[cidx_v.at[j0]], add=True)

        @pl.when(j0 + 2 < CPW)
        def _():
            pltpu.async_copy(y_hbm.at[ridx_v.at[j0 + 2]], rows_v.at[0], sem0)

        pltpu.make_async_copy(y_hbm.at[ridx_v.at[j0 + 1]], rows_v.at[1],
                              sem1).wait()
        pltpu.sync_copy(rows_v.at[1], acc_sh.at[cidx_v.at[j0 + 1]], add=True)

        @pl.when(j0 + 3 < CPW)
        def _():
            pltpu.async_copy(y_hbm.at[ridx_v.at[j0 + 3]], rows_v.at[1], sem1)

        return carry

    lax.fori_loop(0, CPW // 2, step, 0)
    plsc.subcore_barrier()
    pltpu.sync_copy(acc_sh.at[pl.ds(base, RPT)],
                    accs_hbm.at[c, pl.ds(base, RPT)])


_prop_kernel = pl.kernel(
    _prop_body,
    out_type=jax.ShapeDtypeStruct((NC, N_ACC, D), jnp.float32),
    mesh=plsc.VectorSubcoreMesh(core_axis_name="c", subcore_axis_name="s"),
    scratch_types=[
        pltpu.VMEM((CPW, CHUNK), jnp.int32),
        pltpu.VMEM((CPW, CHUNK), jnp.int32),
        pltpu.VMEM((2, CHUNK, D), jnp.float32),
        pltpu.VMEM_SHARED((N_ACC, D), jnp.float32),
        pltpu.SemaphoreType.DMA,
        pltpu.SemaphoreType.DMA,
    ],
)


def _dense_body(x0_ref, wih_ref, whh_ref, bih_ref, bhh_ref, x_ref, hist_ref,
                y_ref, dis_ref, w_scr):
    i = pl.program_id(0)

    @pl.when(i == 0)
    def _():
        x0 = x0_ref[...]
        dn = (((1,), (1,)), ((), ()))
        gi = lax.dot_general(x0, wih_ref[...], dn,
                             preferred_element_type=jnp.float32)
        gi = gi + bih_ref[...]
        gh = lax.dot_general(x0, whh_ref[...], dn,
                             preferred_element_type=jnp.float32)
        gh = gh + bhh_ref[...]
        r = jax.nn.sigmoid(gi[:, 0:D] + gh[:, 0:D])
        z = jax.nn.sigmoid(gi[:, D:2 * D] + gh[:, D:2 * D])
        n = jnp.tanh(gi[:, 2 * D:3 * D] + r * gh[:, 2 * D:3 * D])
        w_scr[...] = (1.0 - z) * n + z * x0

    deg = hist_ref[0, 0, :] + hist_ref[1, 0, :] + 1.0
    dis = lax.rsqrt(deg)
    xw = jnp.dot(x_ref[...], w_scr[...], preferred_element_type=jnp.float32)
    y_ref[...] = dis[:, None] * xw
    dis_ref[0, :] = dis


def _dense(x0, w_ih, w_hh, b_ih, b_hh, x, hist3):
    def full(shape):
        return pl.BlockSpec(shape, lambda i: tuple(0 for _ in shape))

    return pl.pallas_call(
        _dense_body,
        grid=(NB,),
        in_specs=[
            full((D, D)),
            full((3 * D, D)),
            full((3 * D, D)),
            full((1, 3 * D)),
            full((1, 3 * D)),
            pl.BlockSpec((BLK, D), lambda i: (i, 0)),
            pl.BlockSpec((NC, 1, BLK), lambda i: (0, i, 0)),
        ],
        out_specs=[
            pl.BlockSpec((BLK, D), lambda i: (i, 0)),
            pl.BlockSpec((1, BLK), lambda i: (i, 0)),
        ],
        out_shape=[
            jax.ShapeDtypeStruct((N, D), jnp.float32),
            jax.ShapeDtypeStruct((NB, BLK), jnp.float32),
        ],
        scratch_shapes=[pltpu.VMEM((D, D), jnp.float32)],
    )(x0, w_ih, w_hh, b_ih, b_hh, x, hist3)


def _final_body(accs_ref, y_ref, dis_ref, o_ref):
    dis = dis_ref[0, :]
    o_ref[...] = dis[:, None] * (accs_ref[0] + accs_ref[1] + y_ref[...])


def _final(accs, y, dis2):
    return pl.pallas_call(
        _final_body,
        grid=(NB,),
        in_specs=[
            pl.BlockSpec((NC, BLK, D), lambda i: (0, i, 0)),
            pl.BlockSpec((BLK, D), lambda i: (i, 0)),
            pl.BlockSpec((1, BLK), lambda i: (i, 0)),
        ],
        out_specs=pl.BlockSpec((BLK, D), lambda i: (i, 0)),
        out_shape=jax.ShapeDtypeStruct((N, D), jnp.float32),
    )(accs, y, dis2)


def kernel(edge_index, X, initial_weight, W_ih, W_hh, b_ih, b_hh):
    row = edge_index[0]
    col = edge_index[1]
    pad = E_PAD - E
    row3 = jnp.concatenate(
        [row, jnp.zeros((pad,), row.dtype)]).reshape(NW, CPW, CHUNK)
    col3 = jnp.concatenate(
        [col, jnp.full((pad,), N, col.dtype)]).reshape(NW, CPW, CHUNK)

    hist = _deg_kernel(col3)
    hist3 = hist[:, :N, 0].reshape(NC, NB, BLK)

    y, dis2 = _dense(initial_weight[0], W_ih, W_hh,
                     b_ih.reshape(1, 3 * D), b_hh.reshape(1, 3 * D), X, hist3)

    accs = _prop_kernel(row3, col3, y)
    return _final(accs, y, dis2)


# speedup vs baseline: 11.6615x; 11.6615x over previous
import functools

import jax
import jax.numpy as jnp
from jax import lax
from jax.experimental import pallas as pl
from jax.experimental.pallas import tpu as pltpu
from jax.experimental.pallas import tpu_sc as plsc

N = 10000
E = 320000
D = 128
NC = 2
NS = 16
NW = NC * NS
CHUNK = 128
CPW = 80
CPH = CPW // 2
EPW = CHUNK * CPW
E_PAD = NW * EPW
N_ACC = N + 112
RPT = N_ACC // NS
BLK = 1000
NB = N // BLK

_MESH = plsc.VectorSubcoreMesh(core_axis_name="c", subcore_axis_name="s")


HA = 80
HBE = 8000
HNB = E // HBE


def _hist_body(cols_ref, hist_ref):
    i = pl.program_id(0)

    @pl.when(i == 0)
    def _():
        hist_ref[...] = jnp.zeros((HA, 128), jnp.float32)

    contrib = jnp.zeros((HA, 128), jnp.float32)
    for r in range(HBE // 1000):
        colr = cols_ref[r, :]
        a = lax.shift_right_logical(colr, 7)
        b = lax.bitwise_and(colr, 127)
        oa = (a[:, None] == lax.broadcasted_iota(jnp.int32, (1000, HA), 1)
              ).astype(jnp.bfloat16)
        ob = (b[:, None] == lax.broadcasted_iota(jnp.int32, (1000, 128), 1)
              ).astype(jnp.bfloat16)
        contrib += lax.dot_general(oa, ob, (((0,), (0,)), ((), ())),
                                   preferred_element_type=jnp.float32)
    hist_ref[...] += contrib


def _hist(cols2):
    return pl.pallas_call(
        _hist_body,
        grid=(HNB,),
        in_specs=[pl.BlockSpec((HBE // 1000, 1000), lambda i: (i, 0))],
        out_specs=pl.BlockSpec((HA, 128), lambda i: (0, 0)),
        out_shape=jax.ShapeDtypeStruct((HA, 128), jnp.float32),
    )(cols2)


def _prop_body(row_hbm, col_hbm, y_hbm, accs_hbm,
               ridx_v, cidx_v, rows_v, acc_sh, sem0, sem1):
    c = lax.axis_index("c")
    s = lax.axis_index("s")
    wid = s * NC + c

    def fill_zero(i, carry):
        for l in range(D // 16):
            rows_v[0, i, pl.ds(l * 16, 16)] = jnp.zeros((16,), jnp.float32)
        return carry

    lax.fori_loop(0, CHUNK, fill_zero, 0)
    base = s * RPT
    for k in range(RPT // CHUNK):
        pltpu.sync_copy(rows_v.at[0],
                        acc_sh.at[pl.ds(base + k * CHUNK, CHUNK)])
    rem = RPT % CHUNK
    if rem:
        pltpu.sync_copy(rows_v.at[0, pl.ds(0, rem), :],
                        acc_sh.at[pl.ds(base + RPT - rem, rem)])
    plsc.subcore_barrier()

    for h in range(CPW // CPH):
        pltpu.sync_copy(row_hbm.at[wid, pl.ds(h * CPH, CPH)], ridx_v)
        pltpu.sync_copy(col_hbm.at[wid, pl.ds(h * CPH, CPH)], cidx_v)
        pltpu.async_copy(y_hbm.at[ridx_v.at[0]], rows_v.at[0], sem0)
        pltpu.async_copy(y_hbm.at[ridx_v.at[1]], rows_v.at[1], sem1)

        def step(i, carry):
            j0 = 2 * i
            pltpu.make_async_copy(y_hbm.at[ridx_v.at[j0]], rows_v.at[0],
                                  sem0).wait()
            pltpu.sync_copy(rows_v.at[0], acc_sh.at[cidx_v.at[j0]], add=True)

            @pl.when(j0 + 2 < CPH)
            def _():
                pltpu.async_copy(y_hbm.at[ridx_v.at[j0 + 2]], rows_v.at[0],
                                 sem0)

            pltpu.make_async_copy(y_hbm.at[ridx_v.at[j0 + 1]], rows_v.at[1],
                                  sem1).wait()
            pltpu.sync_copy(rows_v.at[1], acc_sh.at[cidx_v.at[j0 + 1]],
                            add=True)

            @pl.when(j0 + 3 < CPH)
            def _():
                pltpu.async_copy(y_hbm.at[ridx_v.at[j0 + 3]], rows_v.at[1],
                                 sem1)

            return carry

        lax.fori_loop(0, CPH // 2, step, 0)

    plsc.subcore_barrier()
    for k in range(RPT // CHUNK):
        pltpu.sync_copy(acc_sh.at[pl.ds(base + k * CHUNK, CHUNK)],
                        rows_v.at[0])
        pltpu.sync_copy(rows_v.at[0],
                        accs_hbm.at[c, pl.ds(base + k * CHUNK, CHUNK)])
    if rem:
        pltpu.sync_copy(acc_sh.at[pl.ds(base + RPT - rem, rem)],
                        rows_v.at[0, pl.ds(0, rem), :])
        pltpu.sync_copy(rows_v.at[0, pl.ds(0, rem), :],
                        accs_hbm.at[c, pl.ds(base + RPT - rem, rem)])


_prop_kernel = pl.kernel(
    _prop_body,
    out_type=jax.ShapeDtypeStruct((NC, N_ACC, D), jnp.float32),
    mesh=_MESH,
    scratch_types=[
        pltpu.VMEM((CPH, CHUNK), jnp.int32),
        pltpu.VMEM((CPH, CHUNK), jnp.int32),
        pltpu.VMEM((2, CHUNK, D), jnp.float32),
        pltpu.VMEM_SHARED((N_ACC, D), jnp.float32),
        pltpu.SemaphoreType.DMA,
        pltpu.SemaphoreType.DMA,
    ],
)


def _dense_body(x0_ref, wih_ref, whh_ref, bih_ref, bhh_ref, x_ref, hist_ref,
                y_ref, dis_ref, w_scr):
    i = pl.program_id(0)

    @pl.when(i == 0)
    def _():
        x0 = x0_ref[...]
        dn = (((1,), (1,)), ((), ()))
        gi = lax.dot_general(x0, wih_ref[...], dn,
                             preferred_element_type=jnp.float32)
        gi = gi + bih_ref[...]
        gh = lax.dot_general(x0, whh_ref[...], dn,
                             preferred_element_type=jnp.float32)
        gh = gh + bhh_ref[...]
        r = jax.nn.sigmoid(gi[:, 0:D] + gh[:, 0:D])
        z = jax.nn.sigmoid(gi[:, D:2 * D] + gh[:, D:2 * D])
        n = jnp.tanh(gi[:, 2 * D:3 * D] + r * gh[:, 2 * D:3 * D])
        w_scr[...] = (1.0 - z) * n + z * x0

    deg = hist_ref[0, 0, :] + 1.0
    dis = lax.rsqrt(deg)
    xw = jnp.dot(x_ref[...], w_scr[...], preferred_element_type=jnp.float32)
    y_ref[...] = dis[:, None] * xw
    dis_ref[...] = dis[:, None]


def _dense(x0, w_ih, w_hh, b_ih, b_hh, x, hist3):
    def full(shape):
        return pl.BlockSpec(shape, lambda i: tuple(0 for _ in shape))

    return pl.pallas_call(
        _dense_body,
        grid=(NB,),
        in_specs=[
            full((D, D)),
            full((3 * D, D)),
            full((3 * D, D)),
            full((1, 3 * D)),
            full((1, 3 * D)),
            pl.BlockSpec((BLK, D), lambda i: (i, 0)),
            pl.BlockSpec((1, 1, BLK), lambda i: (i, 0, 0)),
        ],
        out_specs=[
            pl.BlockSpec((BLK, D), lambda i: (i, 0)),
            pl.BlockSpec((BLK, 1), lambda i: (i, 0)),
        ],
        out_shape=[
            jax.ShapeDtypeStruct((N, D), jnp.float32),
            jax.ShapeDtypeStruct((N, 1), jnp.float32),
        ],
        scratch_shapes=[pltpu.VMEM((D, D), jnp.float32)],
    )(x0, w_ih, w_hh, b_ih, b_hh, x, hist3)


def _final_body(accs_ref, y_ref, dis_ref, o_ref):
    dis = dis_ref[...]
    o_ref[...] = dis * (accs_ref[0] + accs_ref[1] + y_ref[...])


def _final(accs, y, dis2):
    return pl.pallas_call(
        _final_body,
        grid=(NB,),
        in_specs=[
            pl.BlockSpec((NC, BLK, D), lambda i: (0, i, 0)),
            pl.BlockSpec((BLK, D), lambda i: (i, 0)),
            pl.BlockSpec((BLK, 1), lambda i: (i, 0)),
        ],
        out_specs=pl.BlockSpec((BLK, D), lambda i: (i, 0)),
        out_shape=jax.ShapeDtypeStruct((N, D), jnp.float32),
    )(accs, y, dis2)


def kernel(edge_index, X, initial_weight, W_ih, W_hh, b_ih, b_hh):
    row = edge_index[0]
    col = edge_index[1]
    pad = E_PAD - E
    row_p = jnp.concatenate([row, jnp.zeros((pad,), row.dtype)])
    col_p = jnp.concatenate([col, jnp.full((pad,), N, col.dtype)])
    row3 = row_p.reshape(NW, CPW, CHUNK)
    col3 = col_p.reshape(NW, CPW, CHUNK)

    hist = _hist(col.reshape(HNB * (HBE // 1000), 1000))
    hist3 = hist.reshape(HA * 128)[:N].reshape(NB, 1, BLK)

    y, dis2 = _dense(initial_weight[0], W_ih, W_hh,
                     b_ih.reshape(1, 3 * D), b_hh.reshape(1, 3 * D), X, hist3)

    accs = _prop_kernel(row3, col3, y)
    return _final(accs, y, dis2)


# speedup vs baseline: 13.8263x; 1.1856x over previous
import functools

import jax
import jax.numpy as jnp
from jax import lax
from jax.experimental import pallas as pl
from jax.experimental.pallas import tpu as pltpu
from jax.experimental.pallas import tpu_sc as plsc

N = 10000
E = 320000
D = 128
NC = 2
NS = 16
NW = NC * NS
CHUNK = 128
CPW0 = 128
CPW1 = 32
CG = 32
NG0 = CPW0 // CG
NG1 = CPW1 // CG
E_PAD = NS * (CPW0 + CPW1) * CHUNK
N_ACC = N + 112
RPT = N_ACC // NS
BLK = 1000
NB = N // BLK

_MESH = plsc.VectorSubcoreMesh(core_axis_name="c", subcore_axis_name="s")


HA = 80
HBE = 8000
HNB = E // HBE


def _hist_body(cols_ref, hist_ref):
    i = pl.program_id(0)

    @pl.when(i == 0)
    def _():
        hist_ref[...] = jnp.zeros((HA, 128), jnp.float32)

    contrib = jnp.zeros((HA, 128), jnp.float32)
    for r in range(HBE // 1000):
        colr = cols_ref[r, :]
        a = lax.shift_right_logical(colr, 7)
        b = lax.bitwise_and(colr, 127)
        oa = (a[:, None] == lax.broadcasted_iota(jnp.int32, (1000, HA), 1)
              ).astype(jnp.bfloat16)
        ob = (b[:, None] == lax.broadcasted_iota(jnp.int32, (1000, 128), 1)
              ).astype(jnp.bfloat16)
        contrib += lax.dot_general(oa, ob, (((0,), (0,)), ((), ())),
                                   preferred_element_type=jnp.float32)
    hist_ref[...] += contrib


def _hist(cols2):
    return pl.pallas_call(
        _hist_body,
        grid=(HNB,),
        in_specs=[pl.BlockSpec((HBE // 1000, 1000), lambda i: (i, 0))],
        out_specs=pl.BlockSpec((HA, 128), lambda i: (0, 0)),
        out_shape=jax.ShapeDtypeStruct((HA, 128), jnp.float32),
    )(cols2)


def _prop_body(row_hbm, col_hbm, y_hbm, accs_hbm,
               ridx_v, cidx_v, rows_v, acc_sh, sem0, sem1):
    c = lax.axis_index("c")
    s = lax.axis_index("s")
    wid = s * NC + c

    def fill_zero(i, carry):
        for l in range(D // 16):
            rows_v[0, i, pl.ds(l * 16, 16)] = jnp.zeros((16,), jnp.float32)
        return carry

    lax.fori_loop(0, CHUNK, fill_zero, 0)
    base = s * RPT
    for k in range(RPT // CHUNK):
        pltpu.sync_copy(rows_v.at[0],
                        acc_sh.at[pl.ds(base + k * CHUNK, CHUNK)])
    rem = RPT % CHUNK
    if rem:
        pltpu.sync_copy(rows_v.at[0, pl.ds(0, rem), :],
                        acc_sh.at[pl.ds(base + RPT - rem, rem)])
    plsc.subcore_barrier()

    n_groups = jnp.where(c == 0, NG0, NG1)

    def group(g, gcarry):
        pltpu.sync_copy(row_hbm.at[wid, pl.ds(g * CG, CG)], ridx_v)
        pltpu.sync_copy(col_hbm.at[wid, pl.ds(g * CG, CG)], cidx_v)
        pltpu.async_copy(y_hbm.at[ridx_v.at[0]], rows_v.at[0], sem0)
        pltpu.async_copy(y_hbm.at[ridx_v.at[1]], rows_v.at[1], sem1)

        def step(i, carry):
            j0 = 2 * i
            pltpu.make_async_copy(y_hbm.at[ridx_v.at[j0]], rows_v.at[0],
                                  sem0).wait()
            pltpu.sync_copy(rows_v.at[0], acc_sh.at[cidx_v.at[j0]], add=True)

            @pl.when(j0 + 2 < CG)
            def _():
                pltpu.async_copy(y_hbm.at[ridx_v.at[j0 + 2]], rows_v.at[0],
                                 sem0)

            pltpu.make_async_copy(y_hbm.at[ridx_v.at[j0 + 1]], rows_v.at[1],
                                  sem1).wait()
            pltpu.sync_copy(rows_v.at[1], acc_sh.at[cidx_v.at[j0 + 1]],
                            add=True)

            @pl.when(j0 + 3 < CG)
            def _():
                pltpu.async_copy(y_hbm.at[ridx_v.at[j0 + 3]], rows_v.at[1],
                                 sem1)

            return carry

        lax.fori_loop(0, CG // 2, step, 0)
        return gcarry

    lax.fori_loop(0, n_groups, group, 0)

    plsc.subcore_barrier()
    for k in range(RPT // CHUNK):
        pltpu.sync_copy(acc_sh.at[pl.ds(base + k * CHUNK, CHUNK)],
                        rows_v.at[0])
        pltpu.sync_copy(rows_v.at[0],
                        accs_hbm.at[c, pl.ds(base + k * CHUNK, CHUNK)])
    if rem:
        pltpu.sync_copy(acc_sh.at[pl.ds(base + RPT - rem, rem)],
                        rows_v.at[0, pl.ds(0, rem), :])
        pltpu.sync_copy(rows_v.at[0, pl.ds(0, rem), :],
                        accs_hbm.at[c, pl.ds(base + RPT - rem, rem)])


_prop_kernel = pl.kernel(
    _prop_body,
    out_type=jax.ShapeDtypeStruct((NC, N_ACC, D), jnp.float32),
    mesh=_MESH,
    scratch_types=[
        pltpu.VMEM((CG, CHUNK), jnp.int32),
        pltpu.VMEM((CG, CHUNK), jnp.int32),
        pltpu.VMEM((2, CHUNK, D), jnp.float32),
        pltpu.VMEM_SHARED((N_ACC, D), jnp.float32),
        pltpu.SemaphoreType.DMA,
        pltpu.SemaphoreType.DMA,
    ],
)


def _dense_body(x0_ref, wih_ref, whh_ref, bih_ref, bhh_ref, x_ref, hist_ref,
                y_ref, dis_ref, w_scr):
    i = pl.program_id(0)

    @pl.when(i == 0)
    def _():
        x0 = x0_ref[...]
        dn = (((1,), (1,)), ((), ()))
        gi = lax.dot_general(x0, wih_ref[...], dn,
                             preferred_element_type=jnp.float32)
        gi = gi + bih_ref[...]
        gh = lax.dot_general(x0, whh_ref[...], dn,
                             preferred_element_type=jnp.float32)
        gh = gh + bhh_ref[...]
        r = jax.nn.sigmoid(gi[:, 0:D] + gh[:, 0:D])
        z = jax.nn.sigmoid(gi[:, D:2 * D] + gh[:, D:2 * D])
        n = jnp.tanh(gi[:, 2 * D:3 * D] + r * gh[:, 2 * D:3 * D])
        w_scr[...] = (1.0 - z) * n + z * x0

    deg = hist_ref[0, 0, :] + 1.0
    dis = lax.rsqrt(deg)
    xw = jnp.dot(x_ref[...], w_scr[...], preferred_element_type=jnp.float32)
    y_ref[...] = dis[:, None] * xw
    dis_ref[...] = dis[:, None]


def _dense(x0, w_ih, w_hh, b_ih, b_hh, x, hist3):
    def full(shape):
        return pl.BlockSpec(shape, lambda i: tuple(0 for _ in shape))

    return pl.pallas_call(
        _dense_body,
        grid=(NB,),
        in_specs=[
            full((D, D)),
            full((3 * D, D)),
            full((3 * D, D)),
            full((1, 3 * D)),
            full((1, 3 * D)),
            pl.BlockSpec((BLK, D), lambda i: (i, 0)),
            pl.BlockSpec((1, 1, BLK), lambda i: (i, 0, 0)),
        ],
        out_specs=[
            pl.BlockSpec((BLK, D), lambda i: (i, 0)),
            pl.BlockSpec((BLK, 1), lambda i: (i, 0)),
        ],
        out_shape=[
            jax.ShapeDtypeStruct((N, D), jnp.float32),
            jax.ShapeDtypeStruct((N, 1), jnp.float32),
        ],
        scratch_shapes=[pltpu.VMEM((D, D), jnp.float32)],
    )(x0, w_ih, w_hh, b_ih, b_hh, x, hist3)


def _final_body(accs_ref, y_ref, dis_ref, o_ref):
    dis = dis_ref[...]
    o_ref[...] = dis * (accs_ref[0] + accs_ref[1] + y_ref[...])


def _final(accs, y, dis2):
    return pl.pallas_call(
        _final_body,
        grid=(NB,),
        in_specs=[
            pl.BlockSpec((NC, BLK, D), lambda i: (0, i, 0)),
            pl.BlockSpec((BLK, D), lambda i: (i, 0)),
            pl.BlockSpec((BLK, 1), lambda i: (i, 0)),
        ],
        out_specs=pl.BlockSpec((BLK, D), lambda i: (i, 0)),
        out_shape=jax.ShapeDtypeStruct((N, D), jnp.float32),
    )(accs, y, dis2)


def kernel(edge_index, X, initial_weight, W_ih, W_hh, b_ih, b_hh):
    row = edge_index[0]
    col = edge_index[1]
    pad = E_PAD - E
    E0 = NS * CPW0 * CHUNK

    def split(v, pad_val):
        v0 = v[:E0].reshape(NS, CPW0, CHUNK)
        v1 = jnp.concatenate(
            [v[E0:], jnp.full((pad,), pad_val, v.dtype)]).reshape(
                NS, CPW1, CHUNK)
        v1 = jnp.concatenate(
            [v1, jnp.zeros((NS, CPW0 - CPW1, CHUNK), v.dtype)], axis=1)
        return jnp.stack([v0, v1], axis=1).reshape(NW, CPW0, CHUNK)

    row3 = split(row, 0)
    col3 = split(col, N)

    hist = _hist(col.reshape(HNB * (HBE // 1000), 1000))
    hist3 = hist.reshape(HA * 128)[:N].reshape(NB, 1, BLK)

    y, dis2 = _dense(initial_weight[0], W_ih, W_hh,
                     b_ih.reshape(1, 3 * D), b_hh.reshape(1, 3 * D), X, hist3)

    accs = _prop_kernel(row3, col3, y)
    return _final(accs, y, dis2)
